# baseline (speedup 1.0000x reference)
import functools

import jax
import jax.numpy as jnp
from jax import lax
from jax.experimental import pallas as pl
from jax.experimental.pallas import tpu as pltpu
from jax.experimental.pallas import tpu_sc as plsc

B, S, D, E, F = 2, 4096, 768, 8, 3072
T = B * S
C = T // E
EC = E * C
EP = 128
PZ = EC + 8
BT = 512
NB = T // BT
BF = 1536
NF = F // BF
NC, NS = 2, 16
NW = NC * NS
TPW = T // NW
SUB = 64


def _router_body(x_ref, wr_ref, br_ref, dslot_ref, gidx_ref, maxp_ref,
                 cat_ref, cnt_ref, tri_ref):
    i = pl.program_id(0)

    @pl.when(i == 0)
    def _():
        cnt_ref[...] = jnp.zeros_like(cnt_ref)
        rr = lax.broadcasted_iota(jnp.int32, (BT, BT), 0)
        cc = lax.broadcasted_iota(jnp.int32, (BT, BT), 1)
        tri_ref[...] = (rr <= cc).astype(jnp.bfloat16)

    cat_ref[...] = x_ref[...]

    logits = lax.dot_general(
        wr_ref[...].astype(jnp.bfloat16), x_ref[...].astype(jnp.bfloat16),
        (((1,), (1,)), ((), ())), preferred_element_type=jnp.float32)
    logits = logits + br_ref[...]

    m = jnp.max(logits, axis=0, keepdims=True)
    denom = jnp.sum(jnp.exp(logits - m), axis=0, keepdims=True)
    maxp = 1.0 / denom

    ei = lax.broadcasted_iota(jnp.int32, (EP, BT), 0)
    eidx = jnp.min(jnp.where(logits == m, ei, EP), axis=0, keepdims=True)
    onehot = (ei == eidx).astype(jnp.float32)

    csum = lax.dot_general(
        onehot.astype(jnp.bfloat16), tri_ref[...], (((1,), (0,)), ((), ())),
        preferred_element_type=jnp.float32)
    pos_in_block = jnp.sum(csum * onehot, axis=0, keepdims=True)
    prev = jnp.sum(cnt_ref[...] * onehot, axis=0, keepdims=True)
    cnt_ref[...] = cnt_ref[...] + jnp.sum(onehot, axis=1, keepdims=True)

    pos = (prev + pos_in_block - 1.0).astype(jnp.int32)
    keep = pos < C
    slot = eidx * C + pos
    dslot = jnp.where(keep, slot, EC)
    tok = i * BT + lax.broadcasted_iota(jnp.int32, (1, BT), 1)
    gidx = jnp.where(keep, slot, EC + tok)

    dslot_ref[...] = dslot.reshape(1, 1, BT)
    gidx_ref[...] = gidx.reshape(1, 1, BT)
    maxp_ref[...] = maxp.reshape(1, 1, BT)


def _router_call(x, wr_pad, br_pad):
    return pl.pallas_call(
        _router_body,
        grid=(NB,),
        in_specs=[
            pl.BlockSpec((BT, D), lambda i: (i, 0)),
            pl.BlockSpec((EP, D), lambda i: (0, 0)),
            pl.BlockSpec((EP, 1), lambda i: (0, 0)),
        ],
        out_specs=[
            pl.BlockSpec((1, 1, BT), lambda i: (i, 0, 0)),
            pl.BlockSpec((1, 1, BT), lambda i: (i, 0, 0)),
            pl.BlockSpec((1, 1, BT), lambda i: (i, 0, 0)),
            pl.BlockSpec((BT, D), lambda i: (EC // BT + i, 0)),
        ],
        out_shape=[
            jax.ShapeDtypeStruct((NB, 1, BT), jnp.int32),
            jax.ShapeDtypeStruct((NB, 1, BT), jnp.int32),
            jax.ShapeDtypeStruct((NB, 1, BT), jnp.float32),
            jax.ShapeDtypeStruct((EC + T, D), jnp.float32),
        ],
        scratch_shapes=[pltpu.VMEM((EP, 1), jnp.float32),
                        pltpu.VMEM((BT, BT), jnp.bfloat16)],
    )(x, wr_pad, br_pad)


def _dispatch_body(x_hbm, dslot_hbm, maxp_hbm, zeros_hbm, buf_hbm, prob_hbm,
                   xv, dsv, mpv, prob_sp, sem_l0, sem_l1, sem_s0, sem_s1):
    cid = lax.axis_index("c")
    sid = lax.axis_index("s")
    wid = sid * NC + cid
    nchunk = TPW // SUB
    sem_l = (sem_l0, sem_l1)
    sem_s = (sem_s0, sem_s1)

    @pl.when(sid == 0)
    def _():
        pltpu.sync_copy(zeros_hbm, prob_sp)

    plsc.subcore_barrier()

    def start_loads(j):
        b = j % 2
        base = wid * TPW + j * SUB
        return (
            pltpu.async_copy(dslot_hbm.at[pl.ds(base, SUB)], dsv.at[b], sem_l[b]),
            pltpu.async_copy(maxp_hbm.at[pl.ds(base, SUB)], mpv.at[b], sem_l[b]),
            pltpu.async_copy(x_hbm.at[pl.ds(base, SUB)], xv.at[b], sem_l[b]),
        )

    loads = {0: start_loads(0), 1: start_loads(1)}
    scats = {}
    for j in range(nchunk):
        b = j % 2
        for h in loads.pop(j):
            h.wait()
        scats[j] = pltpu.async_copy(xv.at[b], buf_hbm.at[dsv.at[b]], sem_s[b])
        pltpu.sync_copy(mpv.at[b], prob_sp.at[dsv.at[b]], add=True)
        if j + 2 < nchunk:
            scats.pop(j).wait()
            loads[j + 2] = start_loads(j + 2)
    for j in sorted(scats):
        scats.pop(j).wait()

    plsc.subcore_barrier()

    @pl.when(sid == 0)
    def _():
        pltpu.sync_copy(prob_sp, prob_hbm.at[cid])


@functools.cache
def _dispatch_call():
    return pl.kernel(
        _dispatch_body,
        out_type=[
            jax.ShapeDtypeStruct((EC + 8, D), jnp.float32),
            jax.ShapeDtypeStruct((NC, PZ), jnp.float32),
        ],
        mesh=plsc.VectorSubcoreMesh(core_axis_name="c", subcore_axis_name="s"),
        scratch_types=[
            pltpu.VMEM((2, SUB, D), jnp.float32),
            pltpu.VMEM((2, SUB), jnp.int32),
            pltpu.VMEM((2, SUB), jnp.float32),
            pltpu.VMEM_SHARED((PZ,), jnp.float32),
            pltpu.SemaphoreType.DMA,
            pltpu.SemaphoreType.DMA,
            pltpu.SemaphoreType.DMA,
            pltpu.SemaphoreType.DMA,
        ],
    )


def _ffn_body(cat_in_ref, buf_ref, w1_ref, b1_ref, w2_ref, b2_ref, prob_ref,
              out_ref, xin_ref):
    del cat_in_ref
    f = pl.program_id(1)

    @pl.when(f == 0)
    def _():
        xin_ref[...] = buf_ref[...].astype(jnp.bfloat16)

    h = lax.dot_general(
        xin_ref[...], w1_ref[0].astype(jnp.bfloat16), (((1,), (0,)), ((), ())),
        preferred_element_type=jnp.float32)
    h = jnp.maximum(h + b1_ref[...].reshape(1, BF), 0.0).astype(jnp.bfloat16)
    part = lax.dot_general(
        h, w2_ref[0].astype(jnp.bfloat16), (((1,), (0,)), ((), ())),
        preferred_element_type=jnp.float32)

    @pl.when(f == 0)
    def _():
        out_ref[...] = part

    @pl.when(f > 0)
    def _():
        out_ref[...] = out_ref[...] + part

    @pl.when(f == NF - 1)
    def _():
        out_ref[...] = (out_ref[...] + b2_ref[...].reshape(1, D)) * prob_ref[...]


def _ffn_call(cat_init, buf, w1, b1, w2, b2, prob_col):
    return pl.pallas_call(
        _ffn_body,
        grid=(E, NF),
        in_specs=[
            pl.BlockSpec((8, 128), lambda i, f: (0, 0)),
            pl.BlockSpec((C, D), lambda i, f: (i, 0)),
            pl.BlockSpec((1, D, BF), lambda i, f: (i, 0, f)),
            pl.BlockSpec((1, 1, 1, BF), lambda i, f: (i, f, 0, 0)),
            pl.BlockSpec((1, BF, D), lambda i, f: (i, f, 0)),
            pl.BlockSpec((1, 1, D), lambda i, f: (i, 0, 0)),
            pl.BlockSpec((C, 1), lambda i, f: (i, 0)),
        ],
        out_specs=pl.BlockSpec((C, D), lambda i, f: (i, 0)),
        out_shape=jax.ShapeDtypeStruct((EC + T, D), jnp.float32),
        scratch_shapes=[pltpu.VMEM((C, D), jnp.bfloat16)],
        input_output_aliases={0: 0},
    )(cat_init, buf, w1, b1.reshape(E, NF, 1, BF), w2, b2.reshape(E, 1, D),
      prob_col)


def _combine_body(cat_hbm, gidx_hbm, out_hbm, gv, rows,
                  sem_l0, sem_l1, sem_g0, sem_g1, sem_o0, sem_o1):
    cid = lax.axis_index("c")
    sid = lax.axis_index("s")
    wid = sid * NC + cid
    nchunk = TPW // SUB
    sem_l = (sem_l0, sem_l1)
    sem_g = (sem_g0, sem_g1)
    sem_o = (sem_o0, sem_o1)

    def base(j):
        return wid * TPW + j * SUB

    loads = {j: pltpu.async_copy(gidx_hbm.at[pl.ds(base(j), SUB)],
                                 gv.at[j % 2], sem_l[j % 2])
             for j in range(2)}
    stores = {}
    for j in range(nchunk):
        b = j % 2
        loads.pop(j).wait()
        if j - 2 in stores:
            stores.pop(j - 2).wait()
        pltpu.async_copy(cat_hbm.at[gv.at[b]], rows.at[b], sem_g[b]).wait()
        if j + 2 < nchunk:
            loads[j + 2] = pltpu.async_copy(
                gidx_hbm.at[pl.ds(base(j + 2), SUB)], gv.at[b], sem_l[b])
        stores[j] = pltpu.async_copy(rows.at[b], out_hbm.at[pl.ds(base(j), SUB)],
                                     sem_o[b])
    for j in sorted(stores):
        stores.pop(j).wait()


@functools.cache
def _combine_call():
    return pl.kernel(
        _combine_body,
        out_type=jax.ShapeDtypeStruct((T, D), jnp.float32),
        mesh=plsc.VectorSubcoreMesh(core_axis_name="c", subcore_axis_name="s"),
        scratch_types=[
            pltpu.VMEM((2, SUB), jnp.int32),
            pltpu.VMEM((2, SUB, D), jnp.float32),
            pltpu.SemaphoreType.DMA,
            pltpu.SemaphoreType.DMA,
            pltpu.SemaphoreType.DMA,
            pltpu.SemaphoreType.DMA,
            pltpu.SemaphoreType.DMA,
            pltpu.SemaphoreType.DMA,
        ],
    )


def kernel(hidden_states, Wr, br, W1, b1, W2, b2):
    x = hidden_states.reshape(T, D)
    wr_pad = jnp.pad(Wr.T, ((0, EP - E), (0, 0)))
    br_pad = jnp.concatenate(
        [br, jnp.full((EP - E,), -1e30, jnp.float32)]).reshape(EP, 1)

    dslot3, gidx3, maxp3, cat_init = _router_call(x, wr_pad, br_pad)
    dslot = dslot3.reshape(T)
    gidx = gidx3.reshape(T)
    maxp = maxp3.reshape(T)

    zeros = jnp.zeros((PZ,), jnp.float32)
    buf, prob_parts = _dispatch_call()(x, dslot, maxp, zeros)
    prob_col = (prob_parts[0] + prob_parts[1])[:EC, None]

    cat = _ffn_call(cat_init, buf, W1, b1, W2, b2, prob_col)
    out = _combine_call()(cat, gidx)
    return out.reshape(B, S, D)


# speedup vs baseline: 1.0229x; 1.0229x over previous
import functools

import jax
import jax.numpy as jnp
from jax import lax
from jax.experimental import pallas as pl
from jax.experimental.pallas import tpu as pltpu
from jax.experimental.pallas import tpu_sc as plsc

B, S, D, E, F = 2, 4096, 768, 8, 3072
T = B * S
C = T // E
EC = E * C
EP = 128
PZ = EC + 8
BT = 512
NB = T // BT
BF = 1536
NF = F // BF
NC, NS = 2, 16
NW = NC * NS
TPW = T // NW
SUB = 64
CSUB = 128


def _router_body(x_ref, wr_ref, br_ref, dslot_ref, gidx_ref, maxp_ref,
                 cat_ref, cnt_ref, tri_ref):
    i = pl.program_id(0)

    @pl.when(i == 0)
    def _():
        cnt_ref[...] = jnp.zeros_like(cnt_ref)
        rr = lax.broadcasted_iota(jnp.int32, (BT, BT), 0)
        cc = lax.broadcasted_iota(jnp.int32, (BT, BT), 1)
        tri_ref[...] = (rr <= cc).astype(jnp.bfloat16)

    cat_ref[...] = x_ref[...]

    logits = lax.dot_general(
        wr_ref[...].astype(jnp.bfloat16), x_ref[...].astype(jnp.bfloat16),
        (((1,), (1,)), ((), ())), preferred_element_type=jnp.float32)
    logits = logits + br_ref[...]

    m = jnp.max(logits, axis=0, keepdims=True)
    denom = jnp.sum(jnp.exp(logits - m), axis=0, keepdims=True)
    maxp = 1.0 / denom

    ei = lax.broadcasted_iota(jnp.int32, (EP, BT), 0)
    eidx = jnp.min(jnp.where(logits == m, ei, EP), axis=0, keepdims=True)
    onehot = (ei == eidx).astype(jnp.float32)

    csum = lax.dot_general(
        onehot.astype(jnp.bfloat16), tri_ref[...], (((1,), (0,)), ((), ())),
        preferred_element_type=jnp.float32)
    pos_in_block = jnp.sum(csum * onehot, axis=0, keepdims=True)
    prev = jnp.sum(cnt_ref[...] * onehot, axis=0, keepdims=True)
    cnt_ref[...] = cnt_ref[...] + jnp.sum(onehot, axis=1, keepdims=True)

    pos = (prev + pos_in_block - 1.0).astype(jnp.int32)
    keep = pos < C
    slot = eidx * C + pos
    dslot = jnp.where(keep, slot, EC)
    tok = i * BT + lax.broadcasted_iota(jnp.int32, (1, BT), 1)
    gidx = jnp.where(keep, slot, EC + tok)

    dslot_ref[...] = dslot.reshape(1, 1, BT)
    gidx_ref[...] = gidx.reshape(1, 1, BT)
    maxp_ref[...] = maxp.reshape(1, 1, BT)


def _router_call(x, wr_pad, br_pad):
    return pl.pallas_call(
        _router_body,
        grid=(NB,),
        in_specs=[
            pl.BlockSpec((BT, D), lambda i: (i, 0)),
            pl.BlockSpec((EP, D), lambda i: (0, 0)),
            pl.BlockSpec((EP, 1), lambda i: (0, 0)),
        ],
        out_specs=[
            pl.BlockSpec((1, 1, BT), lambda i: (i, 0, 0)),
            pl.BlockSpec((1, 1, BT), lambda i: (i, 0, 0)),
            pl.BlockSpec((1, 1, BT), lambda i: (i, 0, 0)),
            pl.BlockSpec((BT, D), lambda i: (EC // BT + i, 0)),
        ],
        out_shape=[
            jax.ShapeDtypeStruct((NB, 1, BT), jnp.int32),
            jax.ShapeDtypeStruct((NB, 1, BT), jnp.int32),
            jax.ShapeDtypeStruct((NB, 1, BT), jnp.float32),
            jax.ShapeDtypeStruct((EC + T, D), jnp.float32),
        ],
        scratch_shapes=[pltpu.VMEM((EP, 1), jnp.float32),
                        pltpu.VMEM((BT, BT), jnp.bfloat16)],
    )(x, wr_pad, br_pad)


def _dispatch_body(x_hbm, dslot_hbm, maxp_hbm, zeros_hbm, buf_hbm, prob_hbm,
                   xv, dsv, mpv, prob_sp, sem_l0, sem_l1, sem_s0, sem_s1):
    cid = lax.axis_index("c")
    sid = lax.axis_index("s")
    wid = sid * NC + cid
    nchunk = TPW // SUB
    sem_l = (sem_l0, sem_l1)
    sem_s = (sem_s0, sem_s1)

    @pl.when(sid == 0)
    def _():
        pltpu.sync_copy(zeros_hbm, prob_sp)

    plsc.subcore_barrier()

    def start_loads(j):
        b = j % 2
        base = wid * TPW + j * SUB
        return (
            pltpu.async_copy(dslot_hbm.at[pl.ds(base, SUB)], dsv.at[b], sem_l[b]),
            pltpu.async_copy(maxp_hbm.at[pl.ds(base, SUB)], mpv.at[b], sem_l[b]),
            pltpu.async_copy(x_hbm.at[pl.ds(base, SUB)], xv.at[b], sem_l[b]),
        )

    loads = {0: start_loads(0), 1: start_loads(1)}
    scats = {}
    for j in range(nchunk):
        b = j % 2
        for h in loads.pop(j):
            h.wait()
        scats[j] = pltpu.async_copy(xv.at[b], buf_hbm.at[dsv.at[b]], sem_s[b])
        pltpu.sync_copy(mpv.at[b], prob_sp.at[dsv.at[b]], add=True)
        if j + 2 < nchunk:
            scats.pop(j).wait()
            loads[j + 2] = start_loads(j + 2)
    for j in sorted(scats):
        scats.pop(j).wait()

    plsc.subcore_barrier()

    @pl.when(sid == 0)
    def _():
        pltpu.sync_copy(prob_sp, prob_hbm.at[cid])


@functools.cache
def _dispatch_call():
    return pl.kernel(
        _dispatch_body,
        out_type=[
            jax.ShapeDtypeStruct((EC + 8, D), jnp.float32),
            jax.ShapeDtypeStruct((NC, PZ), jnp.float32),
        ],
        mesh=plsc.VectorSubcoreMesh(core_axis_name="c", subcore_axis_name="s"),
        scratch_types=[
            pltpu.VMEM((2, SUB, D), jnp.float32),
            pltpu.VMEM((2, SUB), jnp.int32),
            pltpu.VMEM((2, SUB), jnp.float32),
            pltpu.VMEM_SHARED((PZ,), jnp.float32),
            pltpu.SemaphoreType.DMA,
            pltpu.SemaphoreType.DMA,
            pltpu.SemaphoreType.DMA,
            pltpu.SemaphoreType.DMA,
        ],
    )


def _ffn_body(cat_in_ref, buf_ref, w1_ref, b1_ref, w2_ref, b2_ref, prob_ref,
              out_ref, xin_ref):
    del cat_in_ref
    f = pl.program_id(1)

    @pl.when(f == 0)
    def _():
        xin_ref[...] = buf_ref[...].astype(jnp.bfloat16)

    h = lax.dot_general(
        xin_ref[...], w1_ref[0].astype(jnp.bfloat16), (((1,), (0,)), ((), ())),
        preferred_element_type=jnp.float32)
    h = jnp.maximum(h + b1_ref[...].reshape(1, BF), 0.0).astype(jnp.bfloat16)
    part = lax.dot_general(
        h, w2_ref[0].astype(jnp.bfloat16), (((1,), (0,)), ((), ())),
        preferred_element_type=jnp.float32)

    @pl.when(f == 0)
    def _():
        out_ref[...] = part

    if NF > 2:
        @pl.when((f > 0) & (f < NF - 1))
        def _():
            out_ref[...] = out_ref[...] + part

    @pl.when(f == NF - 1)
    def _():
        out_ref[...] = (out_ref[...] + (part + b2_ref[...].reshape(1, D))) * prob_ref[...]


def _ffn_call(cat_init, buf, w1, b1, w2, b2, prob_col):
    return pl.pallas_call(
        _ffn_body,
        grid=(E, NF),
        in_specs=[
            pl.BlockSpec((8, 128), lambda i, f: (0, 0)),
            pl.BlockSpec((C, D), lambda i, f: (i, 0)),
            pl.BlockSpec((1, D, BF), lambda i, f: (i, 0, f)),
            pl.BlockSpec((1, 1, 1, BF), lambda i, f: (i, f, 0, 0)),
            pl.BlockSpec((1, BF, D), lambda i, f: (i, f, 0)),
            pl.BlockSpec((1, 1, D), lambda i, f: (i, 0, 0)),
            pl.BlockSpec((C, 1), lambda i, f: (i, 0)),
        ],
        out_specs=pl.BlockSpec((C, D), lambda i, f: (i, 0)),
        out_shape=jax.ShapeDtypeStruct((EC + T, D), jnp.float32),
        scratch_shapes=[pltpu.VMEM((C, D), jnp.bfloat16)],
        input_output_aliases={0: 0},
    )(cat_init, buf, w1, b1.reshape(E, NF, 1, BF), w2, b2.reshape(E, 1, D),
      prob_col)


def _combine_body(cat_hbm, gidx_hbm, out_hbm, gv, rows, sem):
    cid = lax.axis_index("c")
    sid = lax.axis_index("s")
    wid = sid * NC + cid
    for sub in range(TPW // CSUB):
        base = wid * TPW + sub * CSUB
        pltpu.sync_copy(gidx_hbm.at[pl.ds(base, CSUB)], gv)
        pltpu.async_copy(cat_hbm.at[gv], rows, sem).wait()
        pltpu.sync_copy(rows, out_hbm.at[pl.ds(base, CSUB)])


@functools.cache
def _combine_call():
    return pl.kernel(
        _combine_body,
        out_type=jax.ShapeDtypeStruct((T, D), jnp.float32),
        mesh=plsc.VectorSubcoreMesh(core_axis_name="c", subcore_axis_name="s"),
        scratch_types=[
            pltpu.VMEM((CSUB,), jnp.int32),
            pltpu.VMEM((CSUB, D), jnp.float32),
            pltpu.SemaphoreType.DMA,
        ],
    )


def kernel(hidden_states, Wr, br, W1, b1, W2, b2):
    x = hidden_states.reshape(T, D)
    wr_pad = jnp.pad(Wr.T, ((0, EP - E), (0, 0)))
    br_pad = jnp.concatenate(
        [br, jnp.full((EP - E,), -1e30, jnp.float32)]).reshape(EP, 1)

    dslot3, gidx3, maxp3, cat_init = _router_call(x, wr_pad, br_pad)
    dslot = dslot3.reshape(T)
    gidx = gidx3.reshape(T)
    maxp = maxp3.reshape(T)

    zeros = jnp.zeros((PZ,), jnp.float32)
    buf, prob_parts = _dispatch_call()(x, dslot, maxp, zeros)
    prob_col = (prob_parts[0] + prob_parts[1])[:EC, None]

    cat = _ffn_call(cat_init, buf, W1, b1, W2, b2, prob_col)
    out = _combine_call()(cat, gidx)
    return out.reshape(B, S, D)
